# baseline (speedup 1.0000x reference)
import functools

import jax
import jax.numpy as jnp
from jax import lax
from jax.experimental import pallas as pl
from jax.experimental.pallas import tpu as pltpu
from jax.experimental.pallas import tpu_sc as plsc

_N = 50000
_NW = 32
_WPT = 1568
_LEVELS = (
    (56, 64, 0.25, 32, 1),
    (28, 128, 0.125, 32, 1),
    (14, 256, 0.0625, 16, 2),
    (7, 512, 1.0 / 32.0, 16, 4),
)


def _scratch_types():
    t = [
        pltpu.VMEM((_WPT,), jnp.float32),
        pltpu.VMEM((_WPT,), jnp.float32),
        pltpu.VMEM((_WPT,), jnp.float32),
        pltpu.VMEM((_WPT,), jnp.float32),
        pltpu.VMEM((_WPT,), jnp.float32),
    ]
    for (_, dim, _, blk, nsub) in _LEVELS:
        t.extend([pltpu.VMEM((blk, dim), jnp.float32)] * 4)
        t.append(pltpu.VMEM((blk, 128), jnp.float32))
        t.extend([pltpu.VMEM((blk,), jnp.int32)] * 4)
        t.extend([pltpu.VMEM((blk,), jnp.float32)] * 4)
    t.append(pltpu.SemaphoreType.DMA)
    return t


def _worker_id():
    return lax.axis_index("s") * 2 + lax.axis_index("c")


def _sc_body(x_hbm, y_hbm, z_hbm, t0, t1, t2, t3, out, *scr):
    xv, yv, zv, hv, wv = scr[:5]
    per_level = []
    k = 5
    for _ in _LEVELS:
        per_level.append(scr[k:k + 13])
        k += 13
    sem = scr[k]

    wid = _worker_id()
    base = jnp.minimum(wid * _WPT, _N - _WPT)

    pltpu.sync_copy(x_hbm.at[pl.ds(base, _WPT)], xv)
    pltpu.sync_copy(y_hbm.at[pl.ds(base, _WPT)], yv)
    pltpu.sync_copy(z_hbm.at[pl.ds(base, _WPT)], zv)

    def hw_body(c, carry):
        s = c * 16
        xx = xv[pl.ds(s, 16)]
        yy = yv[pl.ds(s, 16)]
        zz = zv[pl.ds(s, 16)]
        nz = -zz
        hh = 250.0 * (-yy) / nz + 112.0
        ww = 250.0 * xx / nz + 112.0
        hv[pl.ds(s, 16)] = jnp.minimum(jnp.maximum(hh, 0.0), 223.0)
        wv[pl.ds(s, 16)] = jnp.minimum(jnp.maximum(ww, 0.0), 223.0)
        return carry

    lax.fori_loop(0, _WPT // 16, hw_body, 0)

    tabs = (t0, t1, t2, t3)
    slab_off = []
    acc = 0
    for (_, _, _, _, nsub) in _LEVELS:
        slab_off.append(acc)
        acc += nsub
    for lvl, (H, dim, scale, blk, nsub) in enumerate(_LEVELS):
        tab = tabs[lvl]
        soff = slab_off[lvl]
        lscr = per_level[lvl]
        q11, q21, q12, q22, st = lscr[:5]
        i11, i21, i12, i22, w11, w21, w12, w22 = lscr[5:]
        nb = -(-_WPT // blk)

        def blk_body(b, carry, tab=tab, soff=soff, H=H, dim=dim,
                     scale=scale, blk=blk, nsub=nsub, q11=q11, q21=q21,
                     q12=q12, q22=q22, st=st, i11=i11, i21=i21, i12=i12,
                     i22=i22, w11=w11, w21=w21, w12=w12, w22=w22):
            pb = jnp.minimum(b * blk, _WPT - blk)

            def iw_body(c, carry2):
                s = pb + c * 16
                t = c * 16
                hx = hv[pl.ds(s, 16)] * scale
                wy = wv[pl.ds(s, 16)] * scale
                x1i = hx.astype(jnp.int32)
                x1f = x1i.astype(jnp.float32)
                x2f = jnp.where(x1f == hx, x1f, x1f + 1.0)
                x2i = jnp.minimum(x2f.astype(jnp.int32), H - 1)
                y1i = wy.astype(jnp.int32)
                y1f = y1i.astype(jnp.float32)
                y2f = jnp.where(y1f == wy, y1f, y1f + 1.0)
                y2i = jnp.minimum(y2f.astype(jnp.int32), H - 1)
                dx2 = x2f - hx
                dx1 = hx - x1f
                dy2 = y2f - wy
                dy1 = wy - y1f
                i11[pl.ds(t, 16)] = x1i * H + y1i
                i21[pl.ds(t, 16)] = x2i * H + y1i
                i12[pl.ds(t, 16)] = x1i * H + y2i
                i22[pl.ds(t, 16)] = x2i * H + y2i
                w11[pl.ds(t, 16)] = dx2 * dy2
                w21[pl.ds(t, 16)] = dx1 * dy2
                w12[pl.ds(t, 16)] = dx2 * dy1
                w22[pl.ds(t, 16)] = dx1 * dy1
                return carry2

            lax.fori_loop(0, blk // 16, iw_body, 0)

            c1 = pltpu.async_copy(tab.at[i11], q11, sem)
            c2 = pltpu.async_copy(tab.at[i21], q21, sem)
            c3 = pltpu.async_copy(tab.at[i12], q12, sem)
            c4 = pltpu.async_copy(tab.at[i22], q22, sem)
            c1.wait()
            c2.wait()
            c3.wait()
            c4.wait()

            nchunk = min(8, dim // 16)

            for sub in range(nsub):

                def fma_body(g, carry2, sub=sub):
                    p0 = g * 16
                    a16 = w11[pl.ds(p0, 16)]
                    b16 = w21[pl.ds(p0, 16)]
                    c16 = w12[pl.ds(p0, 16)]
                    d16 = w22[pl.ds(p0, 16)]
                    for j in range(16):
                        a, bw, cw, dw = a16[j], b16[j], c16[j], d16[j]

                        def ch_body(kl, carry3, j=j, a=a, bw=bw, cw=cw,
                                    dw=dw, sub=sub):
                            p = p0 + j
                            d = pl.ds((sub * 8 + kl) * 16, 16)
                            v = (a * q11[p, d] + bw * q21[p, d]
                                 + cw * q12[p, d] + dw * q22[p, d])
                            st[p, pl.ds(kl * 16, 16)] = v
                            return carry3

                        lax.fori_loop(0, nchunk, ch_body, 0)
                    return carry2

                lax.fori_loop(0, blk // 16, fma_body, 0)
                pltpu.sync_copy(
                    st,
                    out.at[pl.ds((soff + sub) * _N + base + pb, blk)])
            return carry

        lax.fori_loop(0, nb, blk_body, 0)


@functools.cache
def _build_sc_kernel():
    mesh = plsc.VectorSubcoreMesh(
        core_axis_name="c", subcore_axis_name="s", num_cores=2, num_subcores=16
    )
    return functools.partial(
        pl.kernel,
        out_type=jax.ShapeDtypeStruct((8 * _N, 128), jnp.float32),
        mesh=mesh,
        scratch_types=_scratch_types(),
        compiler_params=pltpu.CompilerParams(use_tc_tiling_on_sc=False),
    )(_sc_body)


_FB = 400


def _tc_finish_body(c_ref, a0, a1, a2a, a2b, a3a, a3b, a3c, a3d, out_ref):
    out_ref[:, pl.ds(0, 3)] = c_ref[...]
    out_ref[:, pl.ds(3, 64)] = a0[:, :64]
    out_ref[:, pl.ds(67, 128)] = a1[...]
    out_ref[:, pl.ds(195, 128)] = a2a[...]
    out_ref[:, pl.ds(323, 128)] = a2b[...]
    out_ref[:, pl.ds(451, 128)] = a3a[...]
    out_ref[:, pl.ds(579, 128)] = a3b[...]
    out_ref[:, pl.ds(707, 128)] = a3c[...]
    out_ref[:, pl.ds(835, 128)] = a3d[...]


@functools.cache
def _build_tc_finisher():
    nblk = _N // _FB

    def slab_spec(s):
        return pl.BlockSpec((_FB, 128), lambda i, s=s: (s * nblk + i, 0))

    return pl.pallas_call(
        _tc_finish_body,
        grid=(nblk,),
        in_specs=[pl.BlockSpec((_FB, 3), lambda i: (i, 0))] + [
            slab_spec(s) for s in range(8)
        ],
        out_specs=pl.BlockSpec((_FB, 963), lambda i: (i, 0)),
        out_shape=jax.ShapeDtypeStruct((_N, 963), jnp.float32),
    )


def kernel(inputs, img_feat0, img_feat1, img_feat2, img_feat3):
    x = inputs[:, 0]
    y = inputs[:, 1]
    z = inputs[:, 2]
    t0 = img_feat0.reshape(56 * 56, 64)
    t1 = img_feat1.reshape(28 * 28, 128)
    t2 = img_feat2.reshape(14 * 14, 256)
    t3 = img_feat3.reshape(7 * 7, 512)
    big = _build_sc_kernel()(x, y, z, t0, t1, t2, t3)
    return _build_tc_finisher()(inputs, *([big] * 8))
